# baseline (speedup 1.0000x reference)
import functools
import jax
import jax.numpy as jnp
from jax import lax
from jax.experimental import pallas as pl
from jax.experimental.pallas import tpu as pltpu
from jax.experimental.pallas import tpu_sc as plsc

N = 10000
E = 320000
NFEAT = 128
NHID = 16
NCLASS = 40
HEADS = 8
H1 = HEADS * NHID
R1 = 144
R2 = 48

NC, NS, LN = 2, 16, 16
NW = NC * NS
EPT = E // NW
K = 40
NB = EPT // K
NPAD = 10240
RPT = NPAD // NS
ZR = 128

BLK = 2000
f32 = jnp.float32
i32 = jnp.int32


def _head_bcast_mask(rows, cols, transpose=False):
  a = lax.broadcasted_iota(i32, (rows, cols), 0)
  b = lax.broadcasted_iota(i32, (rows, cols), 1)
  if transpose:
    m = a == b // NHID
  else:
    m = a // NHID == b
  return m.astype(f32)


def _phase_a_body(x_ref, w1_ref, as1_ref, ad1_ref, row1_ref, adt_ref):
  h = jnp.dot(x_ref[...], w1_ref[...], preferred_element_type=f32)
  m = _head_bcast_mask(H1, HEADS)
  a_s = jnp.dot(h, as1_ref[...] * m, preferred_element_type=f32)
  a_d = jnp.dot(h, ad1_ref[...] * m, preferred_element_type=f32)
  row1_ref[:, 0:H1] = h
  row1_ref[:, H1:H1 + HEADS] = a_s
  row1_ref[:, H1 + HEADS:R1] = jnp.zeros((BLK, HEADS), f32)
  adt_ref[:, 0:HEADS] = a_d
  adt_ref[:, HEADS:16] = jnp.zeros((BLK, 8), f32)


def _phase_a(x, w1, as1c, ad1c):
  return pl.pallas_call(
      _phase_a_body,
      grid=(N // BLK,),
      in_specs=[
          pl.BlockSpec((BLK, NFEAT), lambda i: (i, 0)),
          pl.BlockSpec((NFEAT, H1), lambda i: (0, 0)),
          pl.BlockSpec((H1, 1), lambda i: (0, 0)),
          pl.BlockSpec((H1, 1), lambda i: (0, 0)),
      ],
      out_specs=[
          pl.BlockSpec((BLK, R1), lambda i: (i, 0)),
          pl.BlockSpec((BLK, 16), lambda i: (i, 0)),
      ],
      out_shape=[
          jax.ShapeDtypeStruct((N, R1), f32),
          jax.ShapeDtypeStruct((N, 16), f32),
      ],
  )(x, w1, as1c, ad1c)


def _leaky_exp(a):
  return jnp.exp(jnp.where(a >= 0, a, a * 0.2))


_BCAST_DNUMS = lax.GatherDimensionNumbers(
    offset_dims=(), collapsed_slice_dims=(0,), start_index_map=(0,))


def _bcast_lane(v, k):
  idx = jnp.full((LN, 1), k, i32)
  return lax.gather(v, idx, _BCAST_DNUMS, slice_sizes=(1,),
                    mode=lax.GatherScatterMode.PROMISE_IN_BOUNDS)


def _sc1_body(src_hbm, dst_hbm, row1_hbm, adt_hbm, out_hbm,
              rows0_v, rows1_v, adr0_v, adr1_v, sidx_v, didx_v, acc,
              sem_r, sem_a, sem_s):
  cid = lax.axis_index("c")
  sid = lax.axis_index("s")
  wid = sid * NC + cid

  def _z(r, _):
    for j in range(R1 // LN):
      rows0_v[r, pl.ds(j * LN, LN)] = jnp.zeros((LN,), f32)
    return 0
  lax.fori_loop(0, K, _z, 0)
  for z in range(RPT // K):
    pltpu.sync_copy(rows0_v, acc.at[pl.ds(sid * RPT + z * K, K)])

  pltpu.sync_copy(src_hbm.at[wid], sidx_v)
  pltpu.sync_copy(dst_hbm.at[wid], didx_v)

  plsc.subcore_barrier()

  lane = lax.iota(i32, LN)
  rows_bufs = (rows0_v, rows1_v)
  adr_bufs = (adr0_v, adr1_v)

  def _fire(b, par):
    rb, ab = rows_bufs[par], adr_bufs[par]
    pltpu.async_copy(row1_hbm.at[sidx_v.at[b]], rb, sem_r)
    pltpu.async_copy(adt_hbm.at[didx_v.at[b]], ab, sem_a)

  def _wait(par):
    rb, ab = rows_bufs[par], adr_bufs[par]
    pltpu.make_async_copy(row1_hbm.at[sidx_v.at[0]], rb, sem_r).wait()
    pltpu.make_async_copy(adt_hbm.at[didx_v.at[0]], ab, sem_a).wait()

  def _wait_scatter(par):
    pltpu.make_async_copy(rows_bufs[par], acc.at[didx_v.at[0]], sem_s).wait()

  _fire(0, 0)

  def _pair(p, _):
    for par in range(2):
      b = p * 2 + par

      @pl.when(b >= 1)
      def _():
        _wait_scatter((par + 1) % 2)

      @pl.when(b + 1 < NB)
      def _():
        _fire(b + 1, (par + 1) % 2)

      _wait(par)
      rb, ab = rows_bufs[par], adr_bufs[par]

      def _edge(i, _):
        for u in range(4):
          r = i * 4 + u
          asr = rb[r, pl.ds(H1, LN)]
          ads = ab[r, :]
          e = jnp.where(lane < HEADS, _leaky_exp(asr + ads), 0.0)
          rb[r, pl.ds(H1, LN)] = e
          for k in range(HEADS):
            bc = _bcast_lane(e, k)
            rb[r, pl.ds(k * LN, LN)] = rb[r, pl.ds(k * LN, LN)] * bc
        return 0
      lax.fori_loop(0, K // 4, _edge, 0)

      pltpu.async_copy(rb, acc.at[didx_v.at[b]], sem_s, add=True)
    return 0

  lax.fori_loop(0, NB // 2, _pair, 0)
  _wait_scatter((NB - 1) % 2)

  plsc.subcore_barrier()
  pltpu.sync_copy(acc.at[pl.ds(sid * RPT, RPT)],
                  out_hbm.at[cid, pl.ds(sid * RPT, RPT)])


def _sc_layer1(src, dst, row1, adt):
  mesh = plsc.VectorSubcoreMesh(core_axis_name="c", subcore_axis_name="s",
                                num_cores=NC, num_subcores=NS)
  kern = functools.partial(
      pl.kernel,
      out_type=jax.ShapeDtypeStruct((NC, NPAD, R1), f32),
      mesh=mesh,
      compiler_params=pltpu.CompilerParams(use_tc_tiling_on_sc=False,
                                           needs_layout_passes=False),
      scratch_types=[
          pltpu.VMEM((K, R1), f32),
          pltpu.VMEM((K, R1), f32),
          pltpu.VMEM((K, 16), f32),
          pltpu.VMEM((K, 16), f32),
          pltpu.VMEM((NB, K), i32),
          pltpu.VMEM((NB, K), i32),
          pltpu.VMEM_SHARED((NPAD, R1), f32),
          pltpu.SemaphoreType.DMA,
          pltpu.SemaphoreType.DMA,
          pltpu.SemaphoreType.DMA,
      ],
  )(_sc1_body)
  return kern(src.reshape(NW, NB, K), dst.reshape(NW, NB, K), row1, adt)


def _sc2_body(src_hbm, dst_hbm, row2_hbm, meta_hbm, out_hbm,
              rows0_v, rows1_v, mr0_v, mr1_v, sidx_v, didx_v, acc,
              sem_r, sem_a, sem_s):
  cid = lax.axis_index("c")
  sid = lax.axis_index("s")
  wid = sid * NC + cid

  def _z(r, _):
    for j in range(R2 // LN):
      rows0_v[r, pl.ds(j * LN, LN)] = jnp.zeros((LN,), f32)
    return 0
  lax.fori_loop(0, K, _z, 0)
  for z in range(RPT // K):
    pltpu.sync_copy(rows0_v, acc.at[pl.ds(sid * RPT + z * K, K)])

  pltpu.sync_copy(src_hbm.at[wid], sidx_v)
  pltpu.sync_copy(dst_hbm.at[wid], didx_v)

  plsc.subcore_barrier()

  lane = lax.iota(i32, LN)
  rows_bufs = (rows0_v, rows1_v)
  mr_bufs = (mr0_v, mr1_v)

  def _fire(b, par):
    rb, mb = rows_bufs[par], mr_bufs[par]
    pltpu.async_copy(row2_hbm.at[sidx_v.at[b]], rb, sem_r)
    pltpu.async_copy(meta_hbm.at[didx_v.at[b]], mb, sem_a)

  def _wait(par):
    rb, mb = rows_bufs[par], mr_bufs[par]
    pltpu.make_async_copy(row2_hbm.at[sidx_v.at[0]], rb, sem_r).wait()
    pltpu.make_async_copy(meta_hbm.at[didx_v.at[0]], mb, sem_a).wait()

  def _wait_scatter(par):
    pltpu.make_async_copy(rows_bufs[par], acc.at[didx_v.at[0]], sem_s).wait()

  _fire(0, 0)

  def _pair(p, _):
    for par in range(2):
      b = p * 2 + par

      @pl.when(b >= 1)
      def _():
        _wait_scatter((par + 1) % 2)

      @pl.when(b + 1 < NB)
      def _():
        _fire(b + 1, (par + 1) % 2)

      _wait(par)
      rb, mb = rows_bufs[par], mr_bufs[par]

      def _edge(i, _):
        for u in range(4):
          r = i * 4 + u
          v2 = rb[r, pl.ds(2 * LN, LN)]
          a1 = _bcast_lane(v2, NCLASS - 2 * LN)
          a2 = _bcast_lane(mb[r, :], 1)
          e = _leaky_exp(a1 + a2)
          rb[r, pl.ds(0, LN)] = rb[r, pl.ds(0, LN)] * e
          rb[r, pl.ds(LN, LN)] = rb[r, pl.ds(LN, LN)] * e
          rb[r, pl.ds(2 * LN, LN)] = jnp.where(lane == NCLASS - 2 * LN,
                                               e, v2 * e)
        return 0
      lax.fori_loop(0, K // 4, _edge, 0)

      pltpu.async_copy(rb, acc.at[didx_v.at[b]], sem_s, add=True)
    return 0

  lax.fori_loop(0, NB // 2, _pair, 0)
  _wait_scatter((NB - 1) % 2)

  plsc.subcore_barrier()
  pltpu.sync_copy(acc.at[pl.ds(sid * RPT, RPT)],
                  out_hbm.at[cid, pl.ds(sid * RPT, RPT)])


def _sc_layer2(src, dst, row2, meta2):
  mesh = plsc.VectorSubcoreMesh(core_axis_name="c", subcore_axis_name="s",
                                num_cores=NC, num_subcores=NS)
  kern = functools.partial(
      pl.kernel,
      out_type=jax.ShapeDtypeStruct((NC, NPAD, R2), f32),
      mesh=mesh,
      compiler_params=pltpu.CompilerParams(use_tc_tiling_on_sc=False,
                                           needs_layout_passes=False),
      scratch_types=[
          pltpu.VMEM((K, R2), f32),
          pltpu.VMEM((K, R2), f32),
          pltpu.VMEM((K, 16), f32),
          pltpu.VMEM((K, 16), f32),
          pltpu.VMEM((NB, K), i32),
          pltpu.VMEM((NB, K), i32),
          pltpu.VMEM_SHARED((NPAD, R2), f32),
          pltpu.SemaphoreType.DMA,
          pltpu.SemaphoreType.DMA,
          pltpu.SemaphoreType.DMA,
      ],
  )(_sc2_body)
  return kern(src.reshape(NW, NB, K), dst.reshape(NW, NB, K), row2, meta2)


def _phase_c_body(p_ref, row1_ref, adt_ref, w2_ref, as2_ref, ad2_ref, b1_ref,
                  feat_ref, row2_ref, meta_ref):
  s = p_ref[0] + p_ref[1]
  h = row1_ref[:, 0:H1]
  a_s = row1_ref[:, H1:H1 + HEADS]
  a_d = adt_ref[:, 0:HEADS]
  e_self = _leaky_exp(a_s + a_d)
  bm = _head_bcast_mask(HEADS, H1, transpose=True)
  num = s[:, 0:H1] + h * jnp.dot(e_self, bm, preferred_element_type=f32)
  den = s[:, H1:H1 + HEADS] + e_self
  denb = jnp.dot(den, bm, preferred_element_type=f32)
  o = num / (denb + 1e-16) + b1_ref[...]
  feat = jnp.where(o > 0, o, jnp.exp(jnp.minimum(o, 0.0)) - 1.0)
  feat_ref[...] = feat
  g = jnp.dot(feat, w2_ref[...], preferred_element_type=f32)
  a_s2 = jnp.dot(g, as2_ref[...], preferred_element_type=f32)
  a_d2 = jnp.dot(g, ad2_ref[...], preferred_element_type=f32)
  row2_ref[:, 0:NCLASS] = g
  row2_ref[:, NCLASS:NCLASS + 1] = a_s2
  row2_ref[:, NCLASS + 1:R2] = jnp.zeros((BLK, R2 - NCLASS - 1), f32)
  meta_ref[:, 0:1] = a_s2
  meta_ref[:, 1:2] = a_d2
  meta_ref[:, 2:16] = jnp.zeros((BLK, 14), f32)


def _phase_c(p1, row1, adt, w2, as2c, ad2c, b1r):
  return pl.pallas_call(
      _phase_c_body,
      grid=(N // BLK,),
      in_specs=[
          pl.BlockSpec((NC, BLK, R1), lambda i: (0, i, 0)),
          pl.BlockSpec((BLK, R1), lambda i: (i, 0)),
          pl.BlockSpec((BLK, 16), lambda i: (i, 0)),
          pl.BlockSpec((H1, NCLASS), lambda i: (0, 0)),
          pl.BlockSpec((NCLASS, 1), lambda i: (0, 0)),
          pl.BlockSpec((NCLASS, 1), lambda i: (0, 0)),
          pl.BlockSpec((1, H1), lambda i: (0, 0)),
      ],
      out_specs=[
          pl.BlockSpec((BLK, H1), lambda i: (i, 0)),
          pl.BlockSpec((BLK, R2), lambda i: (i, 0)),
          pl.BlockSpec((BLK, 16), lambda i: (i, 0)),
      ],
      out_shape=[
          jax.ShapeDtypeStruct((N, H1), f32),
          jax.ShapeDtypeStruct((N, R2), f32),
          jax.ShapeDtypeStruct((N, 16), f32),
      ],
  )(p1, row1, adt, w2, as2c, ad2c, b1r)


def _phase_e_body(p_ref, row2_ref, meta_ref, b2_ref, out_ref):
  s = p_ref[0] + p_ref[1]
  g = row2_ref[:, 0:NCLASS]
  e_self = _leaky_exp(meta_ref[:, 0:1] + meta_ref[:, 1:2])
  num = s[:, 0:NCLASS] + g * e_self
  den = s[:, NCLASS:NCLASS + 1] + e_self
  o = num / (den + 1e-16) + b2_ref[...]
  m = jnp.max(o, axis=1, keepdims=True)
  lse = jnp.log(jnp.sum(jnp.exp(o - m), axis=1, keepdims=True)) + m
  out_ref[...] = o - lse


def _phase_e(p2, row2, meta2, b2r):
  return pl.pallas_call(
      _phase_e_body,
      grid=(N // BLK,),
      in_specs=[
          pl.BlockSpec((NC, BLK, R2), lambda i: (0, i, 0)),
          pl.BlockSpec((BLK, R2), lambda i: (i, 0)),
          pl.BlockSpec((BLK, 16), lambda i: (i, 0)),
          pl.BlockSpec((1, NCLASS), lambda i: (0, 0)),
      ],
      out_specs=pl.BlockSpec((BLK, NCLASS), lambda i: (i, 0)),
      out_shape=jax.ShapeDtypeStruct((N, NCLASS), f32),
  )(p2, row2, meta2, b2r)


@jax.jit
def kernel(x, edge_index, W1, att_src1, att_dst1, b1, W2, att_src2,
           att_dst2, b2):
  src = edge_index[0].astype(i32)
  dst = edge_index[1].astype(i32)
  as1c = att_src1.reshape(H1, 1)
  ad1c = att_dst1.reshape(H1, 1)
  row1, adt = _phase_a(x, W1, as1c, ad1c)
  p1 = _sc_layer1(src, dst, row1, adt)
  feat, row2, meta2 = _phase_c(p1, row1, adt, W2,
                               att_src2.reshape(NCLASS, 1),
                               att_dst2.reshape(NCLASS, 1),
                               b1.reshape(1, H1))
  p2 = _sc_layer2(src, dst, row2, meta2)
  out = _phase_e(p2, row2, meta2, b2.reshape(1, NCLASS))
  return (out, feat)


# speedup vs baseline: 1.0151x; 1.0151x over previous
import functools
import jax
import jax.numpy as jnp
from jax import lax
from jax.experimental import pallas as pl
from jax.experimental.pallas import tpu as pltpu
from jax.experimental.pallas import tpu_sc as plsc

N = 10000
E = 320000
NFEAT = 128
NHID = 16
NCLASS = 40
HEADS = 8
H1 = HEADS * NHID
R1 = 144
R2 = 48

NC, NS, LN = 2, 16, 16
NW = NC * NS
EPT = E // NW
K = 40
NB = EPT // K
K2 = 80
NB2 = EPT // K2
NPAD = 10240
RPT = NPAD // NS
ZR = 128

BLK = 2000
f32 = jnp.float32
i32 = jnp.int32


def _head_bcast_mask(rows, cols, transpose=False):
  a = lax.broadcasted_iota(i32, (rows, cols), 0)
  b = lax.broadcasted_iota(i32, (rows, cols), 1)
  if transpose:
    m = a == b // NHID
  else:
    m = a // NHID == b
  return m.astype(f32)


def _phase_a_body(x_ref, w1_ref, as1_ref, ad1_ref, row1_ref, adt_ref):
  h = jnp.dot(x_ref[...], w1_ref[...], preferred_element_type=f32)
  m = _head_bcast_mask(H1, HEADS)
  a_s = jnp.dot(h, as1_ref[...] * m, preferred_element_type=f32)
  a_d = jnp.dot(h, ad1_ref[...] * m, preferred_element_type=f32)
  row1_ref[:, 0:H1] = h
  row1_ref[:, H1:H1 + HEADS] = a_s
  row1_ref[:, H1 + HEADS:R1] = jnp.zeros((BLK, HEADS), f32)
  adt_ref[:, 0:HEADS] = a_d
  adt_ref[:, HEADS:16] = jnp.zeros((BLK, 8), f32)


def _phase_a(x, w1, as1c, ad1c):
  return pl.pallas_call(
      _phase_a_body,
      grid=(N // BLK,),
      in_specs=[
          pl.BlockSpec((BLK, NFEAT), lambda i: (i, 0)),
          pl.BlockSpec((NFEAT, H1), lambda i: (0, 0)),
          pl.BlockSpec((H1, 1), lambda i: (0, 0)),
          pl.BlockSpec((H1, 1), lambda i: (0, 0)),
      ],
      out_specs=[
          pl.BlockSpec((BLK, R1), lambda i: (i, 0)),
          pl.BlockSpec((BLK, 16), lambda i: (i, 0)),
      ],
      out_shape=[
          jax.ShapeDtypeStruct((N, R1), f32),
          jax.ShapeDtypeStruct((N, 16), f32),
      ],
  )(x, w1, as1c, ad1c)


def _leaky_exp(a):
  return jnp.exp(jnp.where(a >= 0, a, a * 0.2))


_BCAST_DNUMS = lax.GatherDimensionNumbers(
    offset_dims=(), collapsed_slice_dims=(0,), start_index_map=(0,))


def _bcast_lane(v, k):
  idx = jnp.full((LN, 1), k, i32)
  return lax.gather(v, idx, _BCAST_DNUMS, slice_sizes=(1,),
                    mode=lax.GatherScatterMode.PROMISE_IN_BOUNDS)


def _sc1_body(src_hbm, dst_hbm, row1_hbm, adt_hbm, out_hbm,
              rows0_v, rows1_v, adr0_v, adr1_v, sidx_v, didx_v, acc,
              sem_r, sem_a, sem_s):
  cid = lax.axis_index("c")
  sid = lax.axis_index("s")
  wid = sid * NC + cid

  def _z(r, _):
    for j in range(R1 // LN):
      rows0_v[r, pl.ds(j * LN, LN)] = jnp.zeros((LN,), f32)
    return 0
  lax.fori_loop(0, K, _z, 0)
  for z in range(RPT // K):
    pltpu.sync_copy(rows0_v, acc.at[pl.ds(sid * RPT + z * K, K)])

  pltpu.sync_copy(src_hbm.at[wid], sidx_v)
  pltpu.sync_copy(dst_hbm.at[wid], didx_v)

  plsc.subcore_barrier()

  lane = lax.iota(i32, LN)
  rows_bufs = (rows0_v, rows1_v)
  adr_bufs = (adr0_v, adr1_v)

  def _fire(b, par):
    rb, ab = rows_bufs[par], adr_bufs[par]
    pltpu.async_copy(row1_hbm.at[sidx_v.at[b]], rb, sem_r)
    pltpu.async_copy(adt_hbm.at[didx_v.at[b]], ab, sem_a)

  def _wait(par):
    rb, ab = rows_bufs[par], adr_bufs[par]
    pltpu.make_async_copy(row1_hbm.at[sidx_v.at[0]], rb, sem_r).wait()
    pltpu.make_async_copy(adt_hbm.at[didx_v.at[0]], ab, sem_a).wait()

  def _wait_scatter(par):
    pltpu.make_async_copy(rows_bufs[par], acc.at[didx_v.at[0]], sem_s).wait()

  _fire(0, 0)

  def _pair(p, _):
    for par in range(2):
      b = p * 2 + par

      @pl.when(b >= 1)
      def _():
        _wait_scatter((par + 1) % 2)

      @pl.when(b + 1 < NB)
      def _():
        _fire(b + 1, (par + 1) % 2)

      _wait(par)
      rb, ab = rows_bufs[par], adr_bufs[par]

      def _edge(i, _):
        for u in range(4):
          r = i * 4 + u
          asr = rb[r, pl.ds(H1, LN)]
          ads = ab[r, :]
          e = jnp.where(lane < HEADS, _leaky_exp(asr + ads), 0.0)
          rb[r, pl.ds(H1, LN)] = e
          for k in range(HEADS):
            bc = _bcast_lane(e, k)
            rb[r, pl.ds(k * LN, LN)] = rb[r, pl.ds(k * LN, LN)] * bc
        return 0
      lax.fori_loop(0, K // 4, _edge, 0)

      pltpu.async_copy(rb, acc.at[didx_v.at[b]], sem_s, add=True)
    return 0

  lax.fori_loop(0, NB // 2, _pair, 0)
  _wait_scatter((NB - 1) % 2)

  plsc.subcore_barrier()
  pltpu.sync_copy(acc.at[pl.ds(sid * RPT, RPT)],
                  out_hbm.at[cid, pl.ds(sid * RPT, RPT)])


def _sc_layer1(src, dst, row1, adt):
  mesh = plsc.VectorSubcoreMesh(core_axis_name="c", subcore_axis_name="s",
                                num_cores=NC, num_subcores=NS)
  kern = functools.partial(
      pl.kernel,
      out_type=jax.ShapeDtypeStruct((NC, NPAD, R1), f32),
      mesh=mesh,
      compiler_params=pltpu.CompilerParams(use_tc_tiling_on_sc=False,
                                           needs_layout_passes=False),
      scratch_types=[
          pltpu.VMEM((K, R1), f32),
          pltpu.VMEM((K, R1), f32),
          pltpu.VMEM((K, 16), f32),
          pltpu.VMEM((K, 16), f32),
          pltpu.VMEM((NB, K), i32),
          pltpu.VMEM((NB, K), i32),
          pltpu.VMEM_SHARED((NPAD, R1), f32),
          pltpu.SemaphoreType.DMA,
          pltpu.SemaphoreType.DMA,
          pltpu.SemaphoreType.DMA,
      ],
  )(_sc1_body)
  return kern(src.reshape(NW, NB, K), dst.reshape(NW, NB, K), row1, adt)


def _sc2_body(src_hbm, dst_hbm, row2_hbm, meta_hbm, out_hbm,
              rows0_v, rows1_v, mr0_v, mr1_v, sidx_v, didx_v, acc,
              sem_r, sem_a, sem_s):
  cid = lax.axis_index("c")
  sid = lax.axis_index("s")
  wid = sid * NC + cid

  def _z(r, _):
    for j in range(R2 // LN):
      rows0_v[r, pl.ds(j * LN, LN)] = jnp.zeros((LN,), f32)
    return 0
  lax.fori_loop(0, K2, _z, 0)
  for z in range(RPT // K2):
    pltpu.sync_copy(rows0_v, acc.at[pl.ds(sid * RPT + z * K2, K2)])

  pltpu.sync_copy(src_hbm.at[wid], sidx_v)
  pltpu.sync_copy(dst_hbm.at[wid], didx_v)

  plsc.subcore_barrier()

  lane = lax.iota(i32, LN)
  rows_bufs = (rows0_v, rows1_v)
  mr_bufs = (mr0_v, mr1_v)

  def _fire(b, par):
    rb, mb = rows_bufs[par], mr_bufs[par]
    pltpu.async_copy(row2_hbm.at[sidx_v.at[b]], rb, sem_r)
    pltpu.async_copy(meta_hbm.at[didx_v.at[b]], mb, sem_a)

  def _wait(par):
    rb, mb = rows_bufs[par], mr_bufs[par]
    pltpu.make_async_copy(row2_hbm.at[sidx_v.at[0]], rb, sem_r).wait()
    pltpu.make_async_copy(meta_hbm.at[didx_v.at[0]], mb, sem_a).wait()

  def _wait_scatter(par):
    pltpu.make_async_copy(rows_bufs[par], acc.at[didx_v.at[0]], sem_s).wait()

  _fire(0, 0)

  def _pair(p, _):
    for par in range(2):
      b = p * 2 + par

      @pl.when(b >= 1)
      def _():
        _wait_scatter((par + 1) % 2)

      @pl.when(b + 1 < NB2)
      def _():
        _fire(b + 1, (par + 1) % 2)

      _wait(par)
      rb, mb = rows_bufs[par], mr_bufs[par]

      def _edge(i, _):
        for u in range(4):
          r = i * 4 + u
          v2 = rb[r, pl.ds(2 * LN, LN)]
          a1 = _bcast_lane(v2, NCLASS - 2 * LN)
          a2 = _bcast_lane(mb[r, :], 1)
          e = _leaky_exp(a1 + a2)
          rb[r, pl.ds(0, LN)] = rb[r, pl.ds(0, LN)] * e
          rb[r, pl.ds(LN, LN)] = rb[r, pl.ds(LN, LN)] * e
          rb[r, pl.ds(2 * LN, LN)] = jnp.where(lane == NCLASS - 2 * LN,
                                               e, v2 * e)
        return 0
      lax.fori_loop(0, K2 // 4, _edge, 0)

      pltpu.async_copy(rb, acc.at[didx_v.at[b]], sem_s, add=True)
    return 0

  lax.fori_loop(0, NB2 // 2, _pair, 0)

  if NB2 % 2 == 1:
    b = NB2 - 1
    par = b % 2
    _wait_scatter((par + 1) % 2)
    _wait(par)
    rb, mb = rows_bufs[par], mr_bufs[par]

    def _edge_tail(i, _):
      for u in range(4):
        r = i * 4 + u
        v2 = rb[r, pl.ds(2 * LN, LN)]
        a1 = _bcast_lane(v2, NCLASS - 2 * LN)
        a2 = _bcast_lane(mb[r, :], 1)
        e = _leaky_exp(a1 + a2)
        rb[r, pl.ds(0, LN)] = rb[r, pl.ds(0, LN)] * e
        rb[r, pl.ds(LN, LN)] = rb[r, pl.ds(LN, LN)] * e
        rb[r, pl.ds(2 * LN, LN)] = jnp.where(lane == NCLASS - 2 * LN,
                                             e, v2 * e)
      return 0
    lax.fori_loop(0, K2 // 4, _edge_tail, 0)
    pltpu.async_copy(rb, acc.at[didx_v.at[b]], sem_s, add=True)
  _wait_scatter((NB2 - 1) % 2)

  plsc.subcore_barrier()
  pltpu.sync_copy(acc.at[pl.ds(sid * RPT, RPT)],
                  out_hbm.at[cid, pl.ds(sid * RPT, RPT)])


def _sc_layer2(src, dst, row2, meta2):
  mesh = plsc.VectorSubcoreMesh(core_axis_name="c", subcore_axis_name="s",
                                num_cores=NC, num_subcores=NS)
  kern = functools.partial(
      pl.kernel,
      out_type=jax.ShapeDtypeStruct((NC, NPAD, R2), f32),
      mesh=mesh,
      compiler_params=pltpu.CompilerParams(use_tc_tiling_on_sc=False,
                                           needs_layout_passes=False),
      scratch_types=[
          pltpu.VMEM((K2, R2), f32),
          pltpu.VMEM((K2, R2), f32),
          pltpu.VMEM((K2, 16), f32),
          pltpu.VMEM((K2, 16), f32),
          pltpu.VMEM((NB2, K2), i32),
          pltpu.VMEM((NB2, K2), i32),
          pltpu.VMEM_SHARED((NPAD, R2), f32),
          pltpu.SemaphoreType.DMA,
          pltpu.SemaphoreType.DMA,
          pltpu.SemaphoreType.DMA,
      ],
  )(_sc2_body)
  return kern(src.reshape(NW, NB2, K2), dst.reshape(NW, NB2, K2), row2, meta2)


def _phase_c_body(p_ref, row1_ref, adt_ref, w2_ref, as2_ref, ad2_ref, b1_ref,
                  feat_ref, row2_ref, meta_ref):
  s = p_ref[0] + p_ref[1]
  h = row1_ref[:, 0:H1]
  a_s = row1_ref[:, H1:H1 + HEADS]
  a_d = adt_ref[:, 0:HEADS]
  e_self = _leaky_exp(a_s + a_d)
  bm = _head_bcast_mask(HEADS, H1, transpose=True)
  num = s[:, 0:H1] + h * jnp.dot(e_self, bm, preferred_element_type=f32)
  den = s[:, H1:H1 + HEADS] + e_self
  denb = jnp.dot(den, bm, preferred_element_type=f32)
  o = num / (denb + 1e-16) + b1_ref[...]
  feat = jnp.where(o > 0, o, jnp.exp(jnp.minimum(o, 0.0)) - 1.0)
  feat_ref[...] = feat
  g = jnp.dot(feat, w2_ref[...], preferred_element_type=f32)
  a_s2 = jnp.dot(g, as2_ref[...], preferred_element_type=f32)
  a_d2 = jnp.dot(g, ad2_ref[...], preferred_element_type=f32)
  row2_ref[:, 0:NCLASS] = g
  row2_ref[:, NCLASS:NCLASS + 1] = a_s2
  row2_ref[:, NCLASS + 1:R2] = jnp.zeros((BLK, R2 - NCLASS - 1), f32)
  meta_ref[:, 0:1] = a_s2
  meta_ref[:, 1:2] = a_d2
  meta_ref[:, 2:16] = jnp.zeros((BLK, 14), f32)


def _phase_c(p1, row1, adt, w2, as2c, ad2c, b1r):
  return pl.pallas_call(
      _phase_c_body,
      grid=(N // BLK,),
      in_specs=[
          pl.BlockSpec((NC, BLK, R1), lambda i: (0, i, 0)),
          pl.BlockSpec((BLK, R1), lambda i: (i, 0)),
          pl.BlockSpec((BLK, 16), lambda i: (i, 0)),
          pl.BlockSpec((H1, NCLASS), lambda i: (0, 0)),
          pl.BlockSpec((NCLASS, 1), lambda i: (0, 0)),
          pl.BlockSpec((NCLASS, 1), lambda i: (0, 0)),
          pl.BlockSpec((1, H1), lambda i: (0, 0)),
      ],
      out_specs=[
          pl.BlockSpec((BLK, H1), lambda i: (i, 0)),
          pl.BlockSpec((BLK, R2), lambda i: (i, 0)),
          pl.BlockSpec((BLK, 16), lambda i: (i, 0)),
      ],
      out_shape=[
          jax.ShapeDtypeStruct((N, H1), f32),
          jax.ShapeDtypeStruct((N, R2), f32),
          jax.ShapeDtypeStruct((N, 16), f32),
      ],
  )(p1, row1, adt, w2, as2c, ad2c, b1r)


def _phase_e_body(p_ref, row2_ref, meta_ref, b2_ref, out_ref):
  s = p_ref[0] + p_ref[1]
  g = row2_ref[:, 0:NCLASS]
  e_self = _leaky_exp(meta_ref[:, 0:1] + meta_ref[:, 1:2])
  num = s[:, 0:NCLASS] + g * e_self
  den = s[:, NCLASS:NCLASS + 1] + e_self
  o = num / (den + 1e-16) + b2_ref[...]
  m = jnp.max(o, axis=1, keepdims=True)
  lse = jnp.log(jnp.sum(jnp.exp(o - m), axis=1, keepdims=True)) + m
  out_ref[...] = o - lse


def _phase_e(p2, row2, meta2, b2r):
  return pl.pallas_call(
      _phase_e_body,
      grid=(N // BLK,),
      in_specs=[
          pl.BlockSpec((NC, BLK, R2), lambda i: (0, i, 0)),
          pl.BlockSpec((BLK, R2), lambda i: (i, 0)),
          pl.BlockSpec((BLK, 16), lambda i: (i, 0)),
          pl.BlockSpec((1, NCLASS), lambda i: (0, 0)),
      ],
      out_specs=pl.BlockSpec((BLK, NCLASS), lambda i: (i, 0)),
      out_shape=jax.ShapeDtypeStruct((N, NCLASS), f32),
  )(p2, row2, meta2, b2r)


@jax.jit
def kernel(x, edge_index, W1, att_src1, att_dst1, b1, W2, att_src2,
           att_dst2, b2):
  src = edge_index[0].astype(i32)
  dst = edge_index[1].astype(i32)
  as1c = att_src1.reshape(H1, 1)
  ad1c = att_dst1.reshape(H1, 1)
  row1, adt = _phase_a(x, W1, as1c, ad1c)
  p1 = _sc_layer1(src, dst, row1, adt)
  feat, row2, meta2 = _phase_c(p1, row1, adt, W2,
                               att_src2.reshape(NCLASS, 1),
                               att_dst2.reshape(NCLASS, 1),
                               b1.reshape(1, H1))
  p2 = _sc_layer2(src, dst, row2, meta2)
  out = _phase_e(p2, row2, meta2, b2.reshape(1, NCLASS))
  return (out, feat)
